# baseline (speedup 1.0000x reference)
import functools

import jax
import jax.numpy as jnp
from jax import lax
from jax.experimental import pallas as pl
from jax.experimental.pallas import tpu as pltpu
from jax.experimental.pallas import tpu_sc as plsc

INC = 128
INN = 512
OUTC = 64
OUTN = 64
MAXD = 8
N = 512

BN = 64
N_SC = 128
N_TC = N - N_SC

NW = 32
LANES = 16
R_SC = N_SC * INC
ROWS_W = R_SC // NW
CH = 64
CHUNKS = ROWS_W // CH
NGROUPS = OUTN // LANES


def _tc_body(a_ref, m_ref, w_ref, b_ref, x_ref, o_ref, p_scr):
    @pl.when(pl.program_id(0) == 0)
    def _():
        rows = jax.lax.broadcasted_iota(jnp.int32, (INN, OUTN), 0)
        acc = jnp.zeros((INN, OUTN), jnp.float32)
        for d in range(MAXD):
            acc = acc + jnp.where(rows == a_ref[d : d + 1, :],
                                  m_ref[d : d + 1, :], 0.0)
        p_scr[...] = acc.astype(jnp.bfloat16)

    xb = x_ref[...].reshape(BN * INC, INN).astype(jnp.bfloat16)
    xp = jnp.dot(xb, p_scr[...], preferred_element_type=jnp.float32)
    xp = xp.reshape(BN, INC, OUTN)
    yb = jax.lax.dot_general(xp, w_ref[...], (((1,), (0,)), ((), ())),
                             preferred_element_type=jnp.float32)
    o_ref[...] = jnp.transpose(yb, (0, 2, 1)) + b_ref[...][None, :, :]


def _tc_tail_body(y_ref, xp_ref, w_ref, b_ref, o_ref):
    del y_ref
    yb = jax.lax.dot_general(xp_ref[...], w_ref[...], (((1,), (0,)), ((), ())),
                             preferred_element_type=jnp.float32)
    o_ref[...] = jnp.transpose(yb, (0, 2, 1)) + b_ref[...][None, :, :]


_sc_mesh = plsc.VectorSubcoreMesh(core_axis_name="c", subcore_axis_name="s")


@functools.partial(
    pl.kernel,
    mesh=_sc_mesh,
    compiler_params=pltpu.CompilerParams(needs_layout_passes=False),
    out_type=jax.ShapeDtypeStruct((R_SC * OUTN,), jnp.float32),
    scratch_types=[
        pltpu.VMEM((INN,), jnp.int32),
        pltpu.VMEM((INN,), jnp.float32),
        pltpu.VMEM((2, CH, INN), jnp.float32),
        pltpu.VMEM((2 * CH * OUTN,), jnp.float32),
        pltpu.SemaphoreType.DMA,
        pltpu.SemaphoreType.DMA,
        pltpu.SemaphoreType.DMA,
        pltpu.SemaphoreType.DMA,
    ],
)
def _sc_pool(x_hbm, a_flat, m_flat, out_flat,
             a_v, m_v, xbuf, obuf, isem0, isem1, osem0, osem1):
    wid = lax.axis_index("s") * 2 + lax.axis_index("c")
    pltpu.sync_copy(a_flat, a_v)
    pltpu.sync_copy(m_flat, m_v)

    isems = (isem0, isem1)
    osems = (osem0, osem1)
    lanes = lax.broadcasted_iota(jnp.int32, (LANES,), 0)
    itab = []
    mtab = []
    for g in range(NGROUPS):
        pos_g = [lanes * MAXD + (LANES * MAXD * g + d) for d in range(MAXD)]
        itab.append([plsc.load_gather(a_v, [p]) for p in pos_g])
        mtab.append([plsc.load_gather(m_v, [p]) for p in pos_g])

    n0 = N_TC + wid * (ROWS_W // INC)
    per_n = INC // CH

    def start_in(ci):
        buf = ci % 2
        n_i = n0 + ci // per_n
        h = ci % per_n
        return pltpu.async_copy(
            x_hbm.at[n_i, pl.ds(h * CH, CH), :], xbuf.at[buf], isems[buf])

    def start_out(ci):
        buf = ci % 2
        dst = (wid * ROWS_W + ci * CH) * OUTN
        return pltpu.async_copy(
            obuf.at[pl.ds(buf * CH * OUTN, CH * OUTN)],
            out_flat.at[pl.ds(dst, CH * OUTN)], osems[buf])

    in_h = {0: start_in(0)}
    out_h = {}
    for ci in range(CHUNKS):
        buf = ci % 2
        if ci + 1 < CHUNKS:
            in_h[ci + 1] = start_in(ci + 1)
        in_h[ci].wait()
        if ci >= 2:
            out_h[ci - 2].wait()
        cvec = jnp.full((LANES,), buf, dtype=jnp.int32)
        for g in range(NGROUPS):
            ig = itab[g]
            mg = mtab[g]

            def row_fn(r, carry, ig=ig, mg=mg, g=g, buf=buf):
                rvec = jnp.full((LANES,), r, dtype=jnp.int32)
                acc = plsc.load_gather(xbuf, [cvec, rvec, ig[0]]) * mg[0]
                for d in range(1, MAXD):
                    acc = acc + plsc.load_gather(
                        xbuf, [cvec, rvec, ig[d]]) * mg[d]
                obuf[pl.ds(buf * CH * OUTN + r * OUTN + g * LANES, LANES)] = acc
                return carry

            lax.fori_loop(0, CH, row_fn, 0)
        out_h[ci] = start_out(ci)
    for ci in (CHUNKS - 2, CHUNKS - 1):
        if ci >= 0:
            out_h[ci].wait()


@jax.jit
def kernel(x, A, mask, weight, bias):
    at = A.T.astype(jnp.int32)
    mt = mask[:, :, 0].T

    pooled = _sc_pool(x,
                      A.reshape(-1).astype(jnp.int32),
                      mask.reshape(-1))
    pooled = pooled.reshape(N_SC, INC, OUTN)

    y1 = pl.pallas_call(
        _tc_body,
        grid=(N_TC // BN,),
        in_specs=[
            pl.BlockSpec((MAXD, OUTN), lambda i: (0, 0)),
            pl.BlockSpec((MAXD, OUTN), lambda i: (0, 0)),
            pl.BlockSpec((INC, OUTC), lambda i: (0, 0)),
            pl.BlockSpec((OUTC, 1), lambda i: (0, 0)),
            pl.BlockSpec((BN, INC, INN), lambda i: (i, 0, 0)),
        ],
        out_specs=pl.BlockSpec((BN, OUTC, OUTN), lambda i: (i, 0, 0)),
        out_shape=jax.ShapeDtypeStruct((N, OUTC, OUTN), jnp.float32),
        scratch_shapes=[pltpu.VMEM((INN, OUTN), jnp.bfloat16)],
    )(at, mt, weight, bias, x)

    off = N_TC // BN
    y = pl.pallas_call(
        _tc_tail_body,
        grid=(N_SC // BN,),
        in_specs=[
            pl.BlockSpec(memory_space=pltpu.MemorySpace.HBM),
            pl.BlockSpec((BN, INC, OUTN), lambda i: (i, 0, 0)),
            pl.BlockSpec((INC, OUTC), lambda i: (0, 0)),
            pl.BlockSpec((OUTC, 1), lambda i: (0, 0)),
        ],
        out_specs=pl.BlockSpec((BN, OUTC, OUTN), lambda i: (i + off, 0, 0)),
        out_shape=jax.ShapeDtypeStruct((N, OUTC, OUTN), jnp.float32),
        input_output_aliases={0: 0},
    )(y1, pooled, weight, bias)
    return y


# speedup vs baseline: 1.5822x; 1.5822x over previous
import functools

import jax
import jax.numpy as jnp
from jax.experimental import pallas as pl
from jax.experimental.pallas import tpu as pltpu

INC = 128
INN = 512
OUTC = 64
OUTN = 64
MAXD = 8
N = 512

BN = 64


def _body(a_ref, m_ref, w_ref, b_ref, x_ref, o_ref, p_scr):
    @pl.when(pl.program_id(0) == 0)
    def _():
        rows = jax.lax.broadcasted_iota(jnp.int32, (INN, OUTN), 0)
        acc = jnp.zeros((INN, OUTN), jnp.float32)
        for d in range(MAXD):
            acc = acc + jnp.where(rows == a_ref[d : d + 1, :],
                                  m_ref[d : d + 1, :], 0.0)
        p_scr[...] = acc.astype(jnp.bfloat16)

    xb = x_ref[...].reshape(BN * INC, INN).astype(jnp.bfloat16)
    xp = jnp.dot(xb, p_scr[...], preferred_element_type=jnp.float32)
    xp = xp.reshape(BN, INC, OUTN)
    yb = jax.lax.dot_general(xp, w_ref[...], (((1,), (0,)), ((), ())),
                             preferred_element_type=jnp.float32)
    o_ref[...] = jnp.transpose(yb, (0, 2, 1)) + b_ref[...][None, :, :]


@jax.jit
def kernel(x, A, mask, weight, bias):
    at = A.T.astype(jnp.int32)
    mt = mask[:, :, 0].T
    grid = (N // BN,)
    out = pl.pallas_call(
        _body,
        grid=grid,
        in_specs=[
            pl.BlockSpec((MAXD, OUTN), lambda i: (0, 0)),
            pl.BlockSpec((MAXD, OUTN), lambda i: (0, 0)),
            pl.BlockSpec((INC, OUTC), lambda i: (0, 0)),
            pl.BlockSpec((OUTC, 1), lambda i: (0, 0)),
            pl.BlockSpec((BN, INC, INN), lambda i: (i, 0, 0)),
        ],
        out_specs=pl.BlockSpec((BN, OUTC, OUTN), lambda i: (i, 0, 0)),
        out_shape=jax.ShapeDtypeStruct((N, OUTC, OUTN), jnp.float32),
        scratch_shapes=[pltpu.VMEM((INN, OUTN), jnp.bfloat16)],
    )(at, mt, weight, bias, x)
    return out
